# baseline (speedup 1.0000x reference)
import functools

import jax
import jax.numpy as jnp
from jax import lax
from jax.experimental import pallas as pl
from jax.experimental.pallas import tpu as pltpu
from jax.experimental.pallas import tpu_sc as plsc

B, K, D = 4096, 32, 64
NC, NS = 2, 16
NW = NC * NS
BPW = B // NW
GB = 4
GIDX = GB * K
NG = BPW // GB
DW = D // 2

_HI = -65536
_RND = 0x7FFF


def _word_to_f32(w):
    even = plsc.bitcast(w << 16, jnp.float32)
    odd = plsc.bitcast(w & _HI, jnp.float32)
    return even, odd


def _f32_to_word(even, odd):
    e = plsc.bitcast(even, jnp.int32)
    e = e + _RND + ((e >> 16) & 1)
    o = plsc.bitcast(odd, jnp.int32)
    o = o + _RND + ((o >> 16) & 1)
    return lax.shift_right_logical(e, 16) | (o & _HI)


def _body(idx_hbm, sco_hbm, w_hbm, out_hbm,
          idx_v, sco_v, rows_a, rows_b, out_v, sem_a, sem_b):
    wid = lax.axis_index("s") * NC + lax.axis_index("c")
    base = wid * BPW

    pltpu.sync_copy(idx_hbm.at[wid], idx_v)
    pltpu.sync_copy(sco_hbm.at[pl.ds(base, BPW)], sco_v)

    def start(g, buf, sem):
        pltpu.async_copy(w_hbm.at[idx_v.at[g]], buf, sem)

    def wait(g, buf, sem):
        pltpu.make_async_copy(w_hbm.at[idx_v.at[g]], buf, sem).wait()

    def compute(g, rows):
        for i in range(GB):
            b = g * GB + i
            svecs = [sco_v[b, pl.ds(16 * j, 16)] for j in range(K // 16)]
            acc = [jnp.zeros((16,), jnp.float32) for _ in range(4)]
            for k in range(K):
                r = i * K + k
                wlo = rows[r, pl.ds(0, 16)]
                whi = rows[r, pl.ds(16, 16)]
                e0, o0 = _word_to_f32(wlo)
                e1, o1 = _word_to_f32(whi)
                s = svecs[k // 16][k % 16]
                sv = jnp.full((16,), s, jnp.float32)
                acc[0] = acc[0] + sv * e0
                acc[1] = acc[1] + sv * o0
                acc[2] = acc[2] + sv * e1
                acc[3] = acc[3] + sv * o1
            out_v[b, pl.ds(0, 16)] = _f32_to_word(acc[0], acc[1])
            out_v[b, pl.ds(16, 16)] = _f32_to_word(acc[2], acc[3])

    start(0, rows_a, sem_a)
    start(1, rows_b, sem_b)

    def loop_body(g2, carry):
        g = g2 * 2
        wait(g, rows_a, sem_a)
        compute(g, rows_a)

        @pl.when(g + 2 < NG)
        def _():
            start(g + 2, rows_a, sem_a)

        wait(g + 1, rows_b, sem_b)
        compute(g + 1, rows_b)

        @pl.when(g + 3 < NG)
        def _():
            start(g + 3, rows_b, sem_b)

        return carry

    lax.fori_loop(0, NG // 2, loop_body, 0)

    pltpu.sync_copy(out_v, out_hbm.at[pl.ds(base, BPW)])


_sc_call = functools.partial(
    pl.kernel,
    out_type=jax.ShapeDtypeStruct((B, DW), jnp.int32),
    mesh=plsc.VectorSubcoreMesh(
        core_axis_name="c", subcore_axis_name="s",
        num_cores=NC, num_subcores=NS),
    compiler_params=pltpu.CompilerParams(
        needs_layout_passes=False, use_tc_tiling_on_sc=False),
    scratch_types=[
        pltpu.VMEM((NG, GIDX), jnp.int32),
        pltpu.VMEM((BPW, K), jnp.float32),
        pltpu.VMEM((GIDX, DW), jnp.int32),
        pltpu.VMEM((GIDX, DW), jnp.int32),
        pltpu.VMEM((BPW, DW), jnp.int32),
        pltpu.SemaphoreType.DMA,
        pltpu.SemaphoreType.DMA,
    ],
)(_body)


def kernel(indices, scores, weight):
    idx3 = indices.reshape(NW, NG, GIDX)
    u16v = jax.lax.bitcast_convert_type(weight, jnp.uint16)
    ut = u16v.T
    lo = ut[0::2, :].astype(jnp.uint32)
    hi = ut[1::2, :].astype(jnp.uint32)
    tab_t = lo | (hi << 16)
    tab = jax.lax.bitcast_convert_type(tab_t.T, jnp.int32)
    out32 = _sc_call(idx3, scores, tab)
    return jax.lax.bitcast_convert_type(out32, jnp.bfloat16).reshape(B, D)


# speedup vs baseline: 5.7481x; 5.7481x over previous
import functools

import jax
import jax.numpy as jnp
from jax import lax
from jax.experimental import pallas as pl
from jax.experimental.pallas import tpu as pltpu
from jax.experimental.pallas import tpu_sc as plsc

B, K, D = 4096, 32, 64
NC, NS = 2, 16
NW = NC * NS
BPW = B // NW
GB = 4
GIDX = GB * K
NG = BPW // GB
DW = D // 2
V = 1000000
VQ = V // 4
RB = 512

_HI = -65536
_RND = 0x7FFF


def _pack_body(x_ref, o_ref):
    x = x_ref[...].astype(jnp.uint32)
    o_ref[...] = (x[0:32, :] | (x[32:64, :] << 16)).astype(jnp.int32)


_tc_pack = pl.pallas_call(
    _pack_body,
    grid=(pl.cdiv(V, RB),),
    in_specs=[pl.BlockSpec((64, RB), lambda i: (0, i))],
    out_specs=pl.BlockSpec((DW, RB), lambda i: (0, i)),
    out_shape=jax.ShapeDtypeStruct((DW, V), jnp.int32),
)


def _word_to_f32(w):
    even = plsc.bitcast(w << 16, jnp.float32)
    odd = plsc.bitcast(w & _HI, jnp.float32)
    return even, odd


def _f32_to_word(even, odd):
    e = plsc.bitcast(even, jnp.int32)
    e = e + _RND + ((e >> 16) & 1)
    o = plsc.bitcast(odd, jnp.int32)
    o = o + _RND + ((o >> 16) & 1)
    return lax.shift_right_logical(e, 16) | (o & _HI)


def _body(idx_hbm, sco_hbm, w_hbm, out_hbm,
          idx_v, sco_v, rows_a, rows_b, out_v, sem_a, sem_b):
    wid = lax.axis_index("s") * NC + lax.axis_index("c")
    base = wid * BPW

    pltpu.sync_copy(idx_hbm.at[wid], idx_v)
    pltpu.sync_copy(sco_hbm.at[pl.ds(base, BPW)], sco_v)

    def start(g, buf, sem):
        pltpu.async_copy(w_hbm.at[idx_v.at[g]], buf, sem)

    def wait(g, buf, sem):
        pltpu.make_async_copy(w_hbm.at[idx_v.at[g]], buf, sem).wait()

    def compute(g, rows):
        for i in range(GB):
            b = g * GB + i
            svecs = [sco_v[b, pl.ds(16 * j, 16)] for j in range(K // 16)]
            acc = [jnp.zeros((16,), jnp.float32) for _ in range(4)]
            for k in range(K):
                r = i * K + k
                s = svecs[k // 16][k % 16]
                sv = jnp.full((16,), s, jnp.float32)
                wlo = rows[r, pl.ds(0, 16)]
                whi = rows[r, pl.ds(16, 16)]
                e0, o0 = _word_to_f32(wlo)
                e1, o1 = _word_to_f32(whi)
                acc[0] = acc[0] + sv * e0
                acc[1] = acc[1] + sv * o0
                acc[2] = acc[2] + sv * e1
                acc[3] = acc[3] + sv * o1
            out_v[b, pl.ds(0, 16)] = _f32_to_word(acc[0], acc[1])
            out_v[b, pl.ds(16, 16)] = _f32_to_word(acc[2], acc[3])

    start(0, rows_a, sem_a)
    start(1, rows_b, sem_b)

    def loop_body(g2, carry):
        g = g2 * 2
        wait(g, rows_a, sem_a)
        compute(g, rows_a)

        @pl.when(g + 2 < NG)
        def _():
            start(g + 2, rows_a, sem_a)

        wait(g + 1, rows_b, sem_b)
        compute(g + 1, rows_b)

        @pl.when(g + 3 < NG)
        def _():
            start(g + 3, rows_b, sem_b)

        return carry

    lax.fori_loop(0, NG // 2, loop_body, 0)

    pltpu.sync_copy(out_v, out_hbm.at[pl.ds(base, BPW)])


_sc_call = functools.partial(
    pl.kernel,
    out_type=jax.ShapeDtypeStruct((B, DW), jnp.int32),
    mesh=plsc.VectorSubcoreMesh(
        core_axis_name="c", subcore_axis_name="s",
        num_cores=NC, num_subcores=NS),
    compiler_params=pltpu.CompilerParams(
        needs_layout_passes=False, use_tc_tiling_on_sc=False),
    scratch_types=[
        pltpu.VMEM((NG, GIDX), jnp.int32),
        pltpu.VMEM((BPW, K), jnp.float32),
        pltpu.VMEM((GIDX, DW), jnp.int32),
        pltpu.VMEM((GIDX, DW), jnp.int32),
        pltpu.VMEM((BPW, DW), jnp.int32),
        pltpu.SemaphoreType.DMA,
        pltpu.SemaphoreType.DMA,
    ],
)(_body)


def kernel(indices, scores, weight):
    idx3 = indices.reshape(NW, NG, GIDX)
    wt = jax.lax.bitcast_convert_type(weight, jnp.uint16).T
    tab = _tc_pack(wt).T
    out32 = _sc_call(idx3, scores, tab)
    u = jax.lax.bitcast_convert_type(out32, jnp.uint16)
    ub = jnp.concatenate([u[:, :, 0], u[:, :, 1]], axis=1)
    return jax.lax.bitcast_convert_type(ub, jnp.bfloat16)


# speedup vs baseline: 6.6788x; 1.1619x over previous
import functools

import jax
import jax.numpy as jnp
from jax import lax
from jax.experimental import pallas as pl
from jax.experimental.pallas import tpu as pltpu
from jax.experimental.pallas import tpu_sc as plsc

B, K, D = 4096, 32, 64
NC, NS = 2, 16
NW = NC * NS
BPW = B // NW
GB = 4
GIDX = GB * K
NG = BPW // GB
DW = D // 2
V = 1000000
VQ = V // 4
RB = 512

_HI = -65536
_RND = 0x7FFF


def _pack_table(weight):
    u = jax.lax.bitcast_convert_type(weight, jnp.uint16)
    ur = u.reshape(VQ, 4, D)
    lo = ur[:, :, 0:DW].astype(jnp.uint32)
    hi = ur[:, :, DW:D].astype(jnp.uint32)
    t = lo | (hi << 16)
    return jax.lax.bitcast_convert_type(
        t.reshape(VQ, 128), jnp.int32)


def _word_to_f32(w):
    even = plsc.bitcast(w << 16, jnp.float32)
    odd = plsc.bitcast(w & _HI, jnp.float32)
    return even, odd


def _f32_to_word(even, odd):
    e = plsc.bitcast(even, jnp.int32)
    e = e + _RND + ((e >> 16) & 1)
    o = plsc.bitcast(odd, jnp.int32)
    o = o + _RND + ((o >> 16) & 1)
    return lax.shift_right_logical(e, 16) | (o & _HI)


def _body(idx_hbm, sco_hbm, w_hbm, out_hbm,
          idx_v, idxq_v, sco_v, rows_a, rows_b, out_v, sem_a, sem_b):
    wid = lax.axis_index("s") * NC + lax.axis_index("c")
    base = wid * BPW

    pltpu.sync_copy(idx_hbm.at[wid], idx_v)
    pltpu.sync_copy(sco_hbm.at[pl.ds(base, BPW)], sco_v)

    for g in range(NG):
        for t in range(GIDX // 16):
            idxq_v[g, pl.ds(16 * t, 16)] = idx_v[g, pl.ds(16 * t, 16)] >> 2

    def start(g, buf, sem):
        pltpu.async_copy(w_hbm.at[idxq_v.at[g]], buf, sem)

    def wait(g, buf, sem):
        pltpu.make_async_copy(w_hbm.at[idxq_v.at[g]], buf, sem).wait()

    def compute(g, rows):
        for i in range(GB):
            b = g * GB + i
            svecs = [sco_v[b, pl.ds(16 * j, 16)] for j in range(K // 16)]
            ivecs = [idx_v[g, pl.ds(32 * i + 16 * j, 16)] for j in range(K // 16)]
            acc = [jnp.zeros((16,), jnp.float32) for _ in range(4)]
            for k in range(K):
                r = i * K + k
                iv = ivecs[k // 16][k % 16]
                coff = (iv & 3) * DW
                s = svecs[k // 16][k % 16]
                sv = jnp.full((16,), s, jnp.float32)
                wlo = rows[r, pl.ds(coff, 16)]
                whi = rows[r, pl.ds(coff + 16, 16)]
                e0, o0 = _word_to_f32(wlo)
                e1, o1 = _word_to_f32(whi)
                acc[0] = acc[0] + sv * e0
                acc[1] = acc[1] + sv * o0
                acc[2] = acc[2] + sv * e1
                acc[3] = acc[3] + sv * o1
            out_v[b, pl.ds(0, 16)] = _f32_to_word(acc[0], acc[1])
            out_v[b, pl.ds(16, 16)] = _f32_to_word(acc[2], acc[3])

    start(0, rows_a, sem_a)
    start(1, rows_b, sem_b)

    def loop_body(g2, carry):
        g = g2 * 2
        wait(g, rows_a, sem_a)
        compute(g, rows_a)

        @pl.when(g + 2 < NG)
        def _():
            start(g + 2, rows_a, sem_a)

        wait(g + 1, rows_b, sem_b)
        compute(g + 1, rows_b)

        @pl.when(g + 3 < NG)
        def _():
            start(g + 3, rows_b, sem_b)

        return carry

    lax.fori_loop(0, NG // 2, loop_body, 0)

    pltpu.sync_copy(out_v, out_hbm.at[pl.ds(base, BPW)])


_sc_call = functools.partial(
    pl.kernel,
    out_type=jax.ShapeDtypeStruct((B, DW), jnp.int32),
    mesh=plsc.VectorSubcoreMesh(
        core_axis_name="c", subcore_axis_name="s",
        num_cores=NC, num_subcores=NS),
    compiler_params=pltpu.CompilerParams(
        needs_layout_passes=False, use_tc_tiling_on_sc=False),
    scratch_types=[
        pltpu.VMEM((NG, GIDX), jnp.int32),
        pltpu.VMEM((NG, GIDX), jnp.int32),
        pltpu.VMEM((BPW, K), jnp.float32),
        pltpu.VMEM((GIDX, 128), jnp.int32),
        pltpu.VMEM((GIDX, 128), jnp.int32),
        pltpu.VMEM((BPW, DW), jnp.int32),
        pltpu.SemaphoreType.DMA,
        pltpu.SemaphoreType.DMA,
    ],
)(_body)


def kernel(indices, scores, weight):
    idx3 = indices.reshape(NW, NG, GIDX)
    tab = _pack_table(weight)
    out32 = _sc_call(idx3, scores, tab)
    u = jax.lax.bitcast_convert_type(out32, jnp.uint16)
    ub = jnp.concatenate([u[:, :, 0], u[:, :, 1]], axis=1)
    return jax.lax.bitcast_convert_type(ub, jnp.bfloat16)
